# baseline (speedup 1.0000x reference)
import functools

import jax
import jax.numpy as jnp
from jax import lax
from jax.experimental import pallas as pl
from jax.experimental.pallas import tpu as pltpu
from jax.experimental.pallas import tpu_sc as plsc

_BATCH = 16384
_DIM = 128
_V = 1000
_VP = 1024

_SC_ROWS = 12288
_TC_ROWS = _BATCH - _SC_ROWS

_INFO = plsc.get_sparse_core_info()
_NC = _INFO.num_cores
_NS = _INFO.num_subcores
_NW = _NC * _NS
_BPW = _SC_ROWS // _NW
_CHUNK = 64
_NCHUNK = _BPW // _CHUNK

_MESH = plsc.VectorSubcoreMesh(core_axis_name="c", subcore_axis_name="s")


@functools.partial(
    pl.kernel,
    mesh=_MESH,
    out_type=jax.ShapeDtypeStruct((_SC_ROWS, _DIM), jnp.float32),
    scratch_types=[
        pltpu.VMEM((_BPW,), jnp.int32),
        pltpu.VMEM((_BPW, _DIM), jnp.float32),
        pltpu.VMEM_SHARED((_V, _DIM), jnp.float32),
        pltpu.SemaphoreType.DMA((_NCHUNK,)),
        pltpu.SemaphoreType.DMA,
    ],
)
def _gather_rows(idx_hbm, table_hbm, out_hbm, idx_v, rows_v, table_sh, gsems, osem):
    sid = lax.axis_index("s")
    wid = sid * _NC + lax.axis_index("c")
    base = wid * _BPW

    @pl.when(sid == 0)
    def _stage():
        pltpu.sync_copy(table_hbm, table_sh)

    pltpu.sync_copy(idx_hbm.at[pl.ds(base, _BPW)], idx_v)
    plsc.subcore_barrier()
    gathers = [
        pltpu.async_copy(
            table_sh.at[idx_v.at[pl.ds(j * _CHUNK, _CHUNK)]],
            rows_v.at[pl.ds(j * _CHUNK, _CHUNK)],
            gsems.at[j],
        )
        for j in range(_NCHUNK)
    ]
    outs = []
    for j in range(_NCHUNK):
        gathers[j].wait()
        outs.append(
            pltpu.async_copy(
                rows_v.at[pl.ds(j * _CHUNK, _CHUNK)],
                out_hbm.at[pl.ds(base + j * _CHUNK, _CHUNK)],
                osem,
            )
        )
    for o in outs:
        o.wait()


_TC_BLK = 512


def _tc_body(idx_ref, hi_ref, lo_ref, out_ref):
    i = pl.program_id(0)
    idx = idx_ref[pl.ds(i * _TC_BLK, _TC_BLK)]
    oh = (idx[:, None] == lax.broadcasted_iota(jnp.int32, (_TC_BLK, _VP), 1)).astype(jnp.bfloat16)
    acc = jnp.dot(oh, hi_ref[...], preferred_element_type=jnp.float32)
    acc = acc + jnp.dot(oh, lo_ref[...], preferred_element_type=jnp.float32)
    out_ref[...] = acc


def _tc_gather(idx_tc, table):
    tpad = jnp.zeros((_VP, _DIM), jnp.float32).at[:_V].set(table)
    hi = tpad.astype(jnp.bfloat16)
    lo = (tpad - hi.astype(jnp.float32)).astype(jnp.bfloat16)
    return pl.pallas_call(
        _tc_body,
        grid=(_TC_ROWS // _TC_BLK,),
        in_specs=[
            pl.BlockSpec((_TC_ROWS,), lambda i: (0,)),
            pl.BlockSpec((_VP, _DIM), lambda i: (0, 0)),
            pl.BlockSpec((_VP, _DIM), lambda i: (0, 0)),
        ],
        out_specs=pl.BlockSpec((_TC_BLK, _DIM), lambda i: (i, 0)),
        out_shape=jax.ShapeDtypeStruct((_TC_ROWS, _DIM), jnp.float32),
    )(idx_tc, hi, lo)


def kernel(noise_levels, table):
    idx = noise_levels.astype(jnp.int32)
    sc_out = _gather_rows(idx[:_SC_ROWS], table)
    tc_out = _tc_gather(idx[_SC_ROWS:], table)
    return jnp.concatenate([sc_out, tc_out], axis=0)


# speedup vs baseline: 1.3641x; 1.3641x over previous
import functools

import jax
import jax.numpy as jnp
from jax import lax
from jax.experimental import pallas as pl
from jax.experimental.pallas import tpu as pltpu
from jax.experimental.pallas import tpu_sc as plsc

_BATCH = 16384
_DIM = 128

_INFO = plsc.get_sparse_core_info()
_NC = _INFO.num_cores
_NS = _INFO.num_subcores
_NW = _NC * _NS
_BPW = _BATCH // _NW
_CHUNKS = (16, 48, 64, 128, 128, 64, 48, 16)
_OFFS = tuple(sum(_CHUNKS[:j]) for j in range(len(_CHUNKS)))
_NCHUNK = len(_CHUNKS)

_MESH = plsc.VectorSubcoreMesh(core_axis_name="c", subcore_axis_name="s")


@functools.partial(
    pl.kernel,
    mesh=_MESH,
    out_type=jax.ShapeDtypeStruct((_BATCH, _DIM), jnp.float32),
    scratch_types=[
        pltpu.VMEM((_BPW,), jnp.int32),
        pltpu.VMEM((_BPW, _DIM), jnp.float32),
        pltpu.VMEM_SHARED((1000, _DIM), jnp.float32),
        pltpu.SemaphoreType.DMA((_NCHUNK,)),
        pltpu.SemaphoreType.DMA,
        pltpu.SemaphoreType.DMA,
    ],
)
def _gather_rows(idx_hbm, table_hbm, out_hbm, idx_v, rows_v, table_sh, gsems, osem, isem):
    sid = lax.axis_index("s")
    wid = sid * _NC + lax.axis_index("c")
    base = wid * _BPW
    icopy = pltpu.async_copy(idx_hbm.at[pl.ds(base, _BPW)], idx_v, isem)

    @pl.when(sid == 0)
    def _stage():
        pltpu.sync_copy(table_hbm, table_sh)

    icopy.wait()
    plsc.subcore_barrier()
    gathers = [
        pltpu.async_copy(
            table_sh.at[idx_v.at[pl.ds(_OFFS[j], _CHUNKS[j])]],
            rows_v.at[pl.ds(_OFFS[j], _CHUNKS[j])],
            gsems.at[j],
        )
        for j in range(_NCHUNK)
    ]
    outs = []
    for j in range(_NCHUNK):
        gathers[j].wait()
        outs.append(
            pltpu.async_copy(
                rows_v.at[pl.ds(_OFFS[j], _CHUNKS[j])],
                out_hbm.at[pl.ds(base + _OFFS[j], _CHUNKS[j])],
                osem,
            )
        )
    for o in outs:
        o.wait()


def kernel(noise_levels, table):
    idx = noise_levels.astype(jnp.int32)
    return _gather_rows(idx, table)


# speedup vs baseline: 1.3761x; 1.0088x over previous
import functools

import jax
import jax.numpy as jnp
from jax import lax
from jax.experimental import pallas as pl
from jax.experimental.pallas import tpu as pltpu
from jax.experimental.pallas import tpu_sc as plsc

_BATCH = 16384
_DIM = 128

_INFO = plsc.get_sparse_core_info()
_NC = _INFO.num_cores
_NS = _INFO.num_subcores
_NW = _NC * _NS
_BPW = _BATCH // _NW
_CHUNKS = (8, 24, 48, 96, 128, 128, 56, 24)
_OFFS = tuple(sum(_CHUNKS[:j]) for j in range(len(_CHUNKS)))
_NCHUNK = len(_CHUNKS)

_MESH = plsc.VectorSubcoreMesh(core_axis_name="c", subcore_axis_name="s")


@functools.partial(
    pl.kernel,
    mesh=_MESH,
    out_type=jax.ShapeDtypeStruct((_BATCH, _DIM), jnp.float32),
    scratch_types=[
        pltpu.VMEM((_BPW,), jnp.int32),
        pltpu.VMEM((_BPW, _DIM), jnp.float32),
        pltpu.VMEM_SHARED((1000, _DIM), jnp.float32),
        pltpu.SemaphoreType.DMA((_NCHUNK,)),
        pltpu.SemaphoreType.DMA,
        pltpu.SemaphoreType.DMA,
    ],
)
def _gather_rows(idx_hbm, table_hbm, out_hbm, idx_v, rows_v, table_sh, gsems, osem, isem):
    sid = lax.axis_index("s")
    wid = sid * _NC + lax.axis_index("c")
    base = wid * _BPW
    icopy = pltpu.async_copy(idx_hbm.at[pl.ds(base, _BPW)], idx_v, isem)

    @pl.when(sid == 0)
    def _stage():
        pltpu.sync_copy(table_hbm, table_sh)

    icopy.wait()
    plsc.subcore_barrier()
    gathers = [
        pltpu.async_copy(
            table_sh.at[idx_v.at[pl.ds(_OFFS[j], _CHUNKS[j])]],
            rows_v.at[pl.ds(_OFFS[j], _CHUNKS[j])],
            gsems.at[j],
        )
        for j in range(_NCHUNK)
    ]
    outs = []
    for j in range(_NCHUNK):
        gathers[j].wait()
        outs.append(
            pltpu.async_copy(
                rows_v.at[pl.ds(_OFFS[j], _CHUNKS[j])],
                out_hbm.at[pl.ds(base + _OFFS[j], _CHUNKS[j])],
                osem,
            )
        )
    for o in outs:
        o.wait()


def kernel(noise_levels, table):
    idx = noise_levels.astype(jnp.int32)
    return _gather_rows(idx, table)
